# baseline (speedup 1.0000x reference)
import jax
import jax.numpy as jnp
from jax.experimental import pallas as pl
from jax.experimental.pallas import tpu as pltpu

N_SLICES = 8


def _mlp_kernel(x_hbm, w1_ref, b1_ref, w2_ref, b2_ref, w3_ref, b3_ref,
                wr_ref, br_ref, o_ref, xs_ref, sem):
    rows = x_hbm.shape[0] // N_SLICES

    def copy(i):
        return pltpu.make_async_copy(
            x_hbm.at[pl.ds(i * rows, rows)],
            xs_ref.at[pl.ds(i * rows, rows)],
            sem.at[i])

    for i in range(N_SLICES):
        copy(i).start()

    for i in range(N_SLICES):
        copy(i).wait()
        x = xs_ref[pl.ds(i * rows, rows)]
        m = jnp.sum(x, axis=1) * (1.0 / 32.0)
        h = jnp.maximum(
            jnp.dot(m, w1_ref[...], preferred_element_type=jnp.float32)
            + b1_ref[...], 0.0)
        h = jnp.maximum(
            jnp.dot(h, w2_ref[...], preferred_element_type=jnp.float32)
            + b2_ref[...], 0.0)
        h = jnp.maximum(
            jnp.dot(h, w3_ref[...], preferred_element_type=jnp.float32)
            + b3_ref[...], 0.0)
        o_ref[pl.ds(i * rows, rows)] = (
            jnp.dot(h, wr_ref[...], preferred_element_type=jnp.float32)
            + br_ref[...])


def kernel(embeddings, W1, b1, W2, b2, W3, b3, Wr, br):
    B, C, D = embeddings.shape

    vmem = pl.BlockSpec(memory_space=pltpu.MemorySpace.VMEM)
    return pl.pallas_call(
        _mlp_kernel,
        in_specs=[
            pl.BlockSpec(memory_space=pltpu.MemorySpace.HBM),
            vmem, vmem, vmem, vmem, vmem, vmem, vmem, vmem,
        ],
        out_specs=vmem,
        out_shape=jax.ShapeDtypeStruct((B, D), jnp.float32),
        scratch_shapes=[
            pltpu.VMEM((B, C, D), jnp.float32),
            pltpu.SemaphoreType.DMA((N_SLICES,)),
        ],
    )(embeddings, W1, b1, W2, b2, W3, b3, Wr, br)


# speedup vs baseline: 1.4453x; 1.4453x over previous
import jax
import jax.numpy as jnp
from jax.experimental import pallas as pl
from jax.experimental.pallas import tpu as pltpu

N_SLICES = 8


def _mlp_kernel(x_hbm, w1_ref, b1_ref, w2_ref, b2_ref, w3_ref, b3_ref,
                wr_ref, br_ref, o_ref, xs_ref, sem):
    rows = x_hbm.shape[0] // N_SLICES

    def copy(i):
        return pltpu.make_async_copy(
            x_hbm.at[pl.ds(i * rows, rows)],
            xs_ref.at[pl.ds(i * rows, rows)],
            sem.at[i])

    for i in range(N_SLICES):
        copy(i).start()

    for i in range(N_SLICES):
        copy(i).wait()
        x = xs_ref[pl.ds(i * rows, rows)]
        w = x.shape[1]
        while w > 64:
            w //= 2
            x = x[:, :w] + x[:, w:]
        m = x * (1.0 / 32.0)
        h = jnp.maximum(
            jnp.dot(m, w1_ref[...], preferred_element_type=jnp.float32)
            + b1_ref[...], 0.0)
        h = jnp.maximum(
            jnp.dot(h, w2_ref[...], preferred_element_type=jnp.float32)
            + b2_ref[...], 0.0)
        h = jnp.maximum(
            jnp.dot(h, w3_ref[...], preferred_element_type=jnp.float32)
            + b3_ref[...], 0.0)
        o_ref[pl.ds(i * rows, rows)] = (
            jnp.dot(h, wr_ref[...], preferred_element_type=jnp.float32)
            + br_ref[...])


def kernel(embeddings, W1, b1, W2, b2, W3, b3, Wr, br):
    B, C, D = embeddings.shape
    flat = embeddings.reshape(B, C * D)

    vmem = pl.BlockSpec(memory_space=pltpu.MemorySpace.VMEM)
    return pl.pallas_call(
        _mlp_kernel,
        in_specs=[
            pl.BlockSpec(memory_space=pltpu.MemorySpace.HBM),
            vmem, vmem, vmem, vmem, vmem, vmem, vmem, vmem,
        ],
        out_specs=vmem,
        out_shape=jax.ShapeDtypeStruct((B, D), jnp.float32),
        scratch_shapes=[
            pltpu.VMEM((B, C * D), jnp.float32),
            pltpu.SemaphoreType.DMA((N_SLICES,)),
        ],
    )(flat, W1, b1, W2, b2, W3, b3, Wr, br)


# speedup vs baseline: 1.5992x; 1.1065x over previous
import jax
import jax.numpy as jnp
from jax.experimental import pallas as pl

B_BLOCK = 256


def _mlp_kernel(x_ref, w1_ref, b1_ref, w2_ref, b2_ref, w3_ref, b3_ref,
                wr_ref, br_ref, o_ref):
    x = x_ref[...]
    w = x.shape[1]
    while w > 64:
        w //= 2
        x = x[:, :w] + x[:, w:]
    m = x * (1.0 / 32.0)
    h = jnp.maximum(
        jnp.dot(m, w1_ref[...], preferred_element_type=jnp.float32)
        + b1_ref[...], 0.0)
    h = jnp.maximum(
        jnp.dot(h, w2_ref[...], preferred_element_type=jnp.float32)
        + b2_ref[...], 0.0)
    h = jnp.maximum(
        jnp.dot(h, w3_ref[...], preferred_element_type=jnp.float32)
        + b3_ref[...], 0.0)
    o_ref[...] = (
        jnp.dot(h, wr_ref[...], preferred_element_type=jnp.float32)
        + br_ref[...])


def kernel(embeddings, W1, b1, W2, b2, W3, b3, Wr, br):
    B, C, D = embeddings.shape
    H = W1.shape[1]
    grid = (B // B_BLOCK,)
    flat = embeddings.reshape(B, C * D)

    def full(shape):
        return pl.BlockSpec(shape, lambda i: (0,) * len(shape))

    return pl.pallas_call(
        _mlp_kernel,
        grid=grid,
        in_specs=[
            pl.BlockSpec((B_BLOCK, C * D), lambda i: (i, 0)),
            full((D, H)), full((H,)),
            full((H, H)), full((H,)),
            full((H, H)), full((H,)),
            full((H, D)), full((D,)),
        ],
        out_specs=pl.BlockSpec((B_BLOCK, D), lambda i: (i, 0)),
        out_shape=jax.ShapeDtypeStruct((B, D), jnp.float32),
    )(flat, W1, b1, W2, b2, W3, b3, Wr, br)


# speedup vs baseline: 1.6904x; 1.0571x over previous
import jax
import jax.numpy as jnp
from jax.experimental import pallas as pl

B_BLOCK = 1024


def _mlp_kernel(x_ref, w1_ref, b1_ref, w2_ref, b2_ref, w3_ref, b3_ref,
                wr_ref, br_ref, o_ref):
    x = x_ref[...]
    w = x.shape[1]
    while w > 64:
        w //= 2
        x = x[:, :w] + x[:, w:]
    m = x * (1.0 / 32.0)
    h = jnp.maximum(
        jnp.dot(m, w1_ref[...], preferred_element_type=jnp.float32)
        + b1_ref[...], 0.0)
    h = jnp.maximum(
        jnp.dot(h, w2_ref[...], preferred_element_type=jnp.float32)
        + b2_ref[...], 0.0)
    h = jnp.maximum(
        jnp.dot(h, w3_ref[...], preferred_element_type=jnp.float32)
        + b3_ref[...], 0.0)
    o_ref[...] = (
        jnp.dot(h, wr_ref[...], preferred_element_type=jnp.float32)
        + br_ref[...])


def kernel(embeddings, W1, b1, W2, b2, W3, b3, Wr, br):
    B, C, D = embeddings.shape
    H = W1.shape[1]
    grid = (B // B_BLOCK,)
    flat = embeddings.reshape(B, C * D)

    def full(shape):
        return pl.BlockSpec(shape, lambda i: (0,) * len(shape))

    return pl.pallas_call(
        _mlp_kernel,
        grid=grid,
        in_specs=[
            pl.BlockSpec((B_BLOCK, C * D), lambda i: (i, 0)),
            full((D, H)), full((H,)),
            full((H, H)), full((H,)),
            full((H, H)), full((H,)),
            full((H, D)), full((D,)),
        ],
        out_specs=pl.BlockSpec((B_BLOCK, D), lambda i: (i, 0)),
        out_shape=jax.ShapeDtypeStruct((B, D), jnp.float32),
    )(flat, W1, b1, W2, b2, W3, b3, Wr, br)
